# baseline (speedup 1.0000x reference)
import functools

import jax
import jax.numpy as jnp
from jax import lax
from jax.experimental import pallas as pl
from jax.experimental.pallas import tpu as pltpu
from jax.experimental.pallas import tpu_sc as plsc

_BUCKETS = 1000000
_DIM = 64
_MODEL_DIM = 1024
_N = 16384
_NC, _NS = 2, 16
_NW = _NC * _NS
_BPW = _N // _NW
_PR = _BUCKETS // 2
_TC = 128
_NCHT = _BUCKETS // _TC
_TAIL0 = _NCHT * _TC
_CHUNKS_LO = _NCHT // _NW
_EXTRA = _NCHT - _CHUNKS_LO * _NW
_GC = 128

_mesh = plsc.VectorSubcoreMesh(core_axis_name="c", subcore_axis_name="s")


def _wid():
    return (lax.axis_index("s") * jnp.int32(_NC)
            + lax.axis_index("c")).astype(jnp.int32)


@functools.partial(
    pl.kernel,
    mesh=_mesh,
    compiler_params=pltpu.CompilerParams(needs_layout_passes=False),
    out_type=jax.ShapeDtypeStruct((_PR, 2 * _DIM), jnp.float32),
    scratch_types=[
        pltpu.VMEM((_DIM, _TC), jnp.float32),
        pltpu.VMEM((_TC // 2, 2 * _DIM), jnp.float32),
        pltpu.VMEM((_DIM, _TC // 2), jnp.float32),
    ],
)
def _pack(tab_t_hbm, packed_hbm, slab_v, prow_v, tail_v):
    wid = _wid()
    base = wid * jnp.int32(_CHUNKS_LO) + jnp.minimum(wid, jnp.int32(_EXTRA))
    count = jnp.int32(_CHUNKS_LO) + (wid < jnp.int32(_EXTRA)).astype(jnp.int32)
    lane = lax.iota(jnp.int32, 16)

    def _transpose(src_v, rows, dst_row0):
        for p in range(rows):
            for half in range(2):
                t16 = jnp.zeros((16,), jnp.int32) + jnp.int32(2 * p + half)
                for k in range(_DIM // 16):
                    v = plsc.load_gather(
                        src_v, [lane + jnp.int32(16 * k), t16])
                    prow_v[p, pl.ds(half * _DIM + 16 * k, 16)] = v
        pltpu.sync_copy(prow_v.at[pl.ds(0, rows)],
                        packed_hbm.at[pl.ds(dst_row0, rows)])

    def _chunk(i, _):
        c0 = (base + i) * jnp.int32(_TC)
        pltpu.sync_copy(tab_t_hbm.at[:, pl.ds(c0, _TC)], slab_v)
        _transpose(slab_v, _TC // 2, (base + i) * jnp.int32(_TC // 2))
        return _

    lax.fori_loop(jnp.int32(0), count, _chunk, jnp.int32(0))

    @pl.when(wid == jnp.int32(_NW - 1))
    def _tail():
        pltpu.sync_copy(tab_t_hbm.at[:, pl.ds(_TAIL0, _TC // 2)], tail_v)
        _transpose(tail_v, _TC // 4, jnp.int32(_TAIL0 // 2))


@functools.partial(
    pl.kernel,
    mesh=_mesh,
    compiler_params=pltpu.CompilerParams(needs_layout_passes=False),
    out_type=jax.ShapeDtypeStruct((_N, 2 * _DIM), jnp.float32),
    scratch_types=[
        pltpu.VMEM((_BPW,), jnp.int32),
        pltpu.VMEM((_BPW,), jnp.int32),
        pltpu.VMEM((_BPW,), jnp.int32),
        pltpu.VMEM((_BPW,), jnp.int32),
        pltpu.VMEM((_GC, 2 * _DIM), jnp.float32),
        pltpu.VMEM((_BPW, 2 * _DIM), jnp.float32),
        pltpu.SemaphoreType.DMA,
    ],
)
def _hash_gather(ids_hbm, prev_hbm, packed_hbm, out_hbm,
                 ids_v, prev_v, idx_v, sub_v, pair_v, rows_v, sem):
    base = _wid() * jnp.int32(_BPW)
    pltpu.sync_copy(ids_hbm.at[pl.ds(base, _BPW)], ids_v)
    pltpu.sync_copy(prev_hbm.at[pl.ds(base, _BPW)], prev_v)
    for i in range(_BPW // 16):
        x = ids_v[pl.ds(i * 16, 16)]
        p = prev_v[pl.ds(i * 16, 16)]
        t = ((p >> jnp.int32(10)) * jnp.int32(48704)
             + (p & jnp.int32(1023)) * jnp.int32(92821) + x)
        h = t % jnp.int32(_BUCKETS)
        idx_v[pl.ds(i * 16, 16)] = h >> jnp.int32(1)
        sub_v[pl.ds(i * 16, 16)] = h & jnp.int32(1)
    lane = lax.iota(jnp.int32, 16)
    for chunk in range(_BPW // _GC):
        pltpu.async_copy(
            packed_hbm.at[idx_v.at[pl.ds(chunk * _GC, _GC)]],
            pair_v, sem).wait()

        def _extract(c, _):
            c16 = jnp.zeros((16,), jnp.int32) + c
            for g in range(_GC // 16):
                w16 = lane + jnp.int32(g * 16)
                t16 = w16 + jnp.int32(chunk * _GC)
                s16 = sub_v[pl.ds(chunk * _GC + g * 16, 16)]
                v = plsc.load_gather(pair_v,
                                     [w16, s16 * jnp.int32(_DIM) + c16])
                plsc.store_scatter(rows_v, [t16, c16], v)
            return _

        lax.fori_loop(jnp.int32(0), jnp.int32(_DIM), _extract,
                      jnp.int32(0))
    pltpu.sync_copy(rows_v, out_hbm.at[pl.ds(base, _BPW)])


def _mm_body(x_ref, wt_ref, o_ref):
    o_ref[...] = lax.dot_general(
        x_ref[:, :_DIM], wt_ref[...], (((1,), (0,)), ((), ())),
        preferred_element_type=jnp.float32)


_MB = 1024

_mm = pl.pallas_call(
    _mm_body,
    grid=(_N // _MB,),
    in_specs=[
        pl.BlockSpec((_MB, 2 * _DIM), lambda i: (i, jnp.int32(0))),
        pl.BlockSpec((_DIM, _MODEL_DIM),
                     lambda i: (jnp.int32(0), jnp.int32(0))),
    ],
    out_specs=pl.BlockSpec((_MB, _MODEL_DIM), lambda i: (i, jnp.int32(0))),
    out_shape=jax.ShapeDtypeStruct((_N, _MODEL_DIM), jnp.float32),
)


def kernel(input_ids, table, w_proj):
    b, s = input_ids.shape
    ids = input_ids.astype(jnp.int32)
    prev = jnp.pad(ids[:, :-1], ((0, 0), (1, 0)))
    packed = _pack(table.astype(jnp.float32).T)
    emb = _hash_gather(ids.reshape(-1), prev.reshape(-1), packed)
    out = _mm(emb, w_proj.astype(jnp.float32).T)
    return out.astype(w_proj.dtype).reshape(b, s, _MODEL_DIM)


# speedup vs baseline: 1.8025x; 1.8025x over previous
import functools

import jax
import jax.numpy as jnp
from jax import lax
from jax.experimental import pallas as pl
from jax.experimental.pallas import tpu as pltpu
from jax.experimental.pallas import tpu_sc as plsc

_BUCKETS = 1000000
_DIM = 64
_MODEL_DIM = 1024
_N = 16384
_NC, _NS = 2, 16
_NW = _NC * _NS
_BPW = _N // _NW
_CH = 128

_mesh = plsc.VectorSubcoreMesh(core_axis_name="c", subcore_axis_name="s")


@functools.partial(
    pl.kernel,
    mesh=_mesh,
    compiler_params=pltpu.CompilerParams(use_tc_tiling_on_sc=False),
    out_type=jax.ShapeDtypeStruct((_N, _DIM), jnp.float32),
    scratch_types=[
        pltpu.VMEM((_BPW,), jnp.int32),
        pltpu.VMEM((_BPW,), jnp.int32),
        pltpu.VMEM((_BPW,), jnp.int32),
        pltpu.VMEM((_BPW, _DIM), jnp.float32),
        pltpu.SemaphoreType.DMA,
    ],
)
def _hash_gather(ids_hbm, prev_hbm, table_hbm, out_hbm,
                 ids_v, prev_v, idx_v, rows_v, sem):
    wid = (lax.axis_index("s") * jnp.int32(_NC)
           + lax.axis_index("c")).astype(jnp.int32)
    base = wid * jnp.int32(_BPW)
    pltpu.sync_copy(ids_hbm.at[pl.ds(base, _BPW)], ids_v)
    pltpu.sync_copy(prev_hbm.at[pl.ds(base, _BPW)], prev_v)
    for i in range(_BPW // 16):
        x = ids_v[pl.ds(i * 16, 16)]
        p = prev_v[pl.ds(i * 16, 16)]
        t = ((p >> jnp.int32(10)) * jnp.int32(48704)
             + (p & jnp.int32(1023)) * jnp.int32(92821) + x)
        idx_v[pl.ds(i * 16, 16)] = t % jnp.int32(_BUCKETS)
    copies = [
        pltpu.async_copy(
            table_hbm.at[idx_v.at[pl.ds(j * _CH, _CH)]],
            rows_v.at[pl.ds(j * _CH, _CH)],
            sem,
        )
        for j in range(_BPW // _CH)
    ]
    for c in copies:
        c.wait()
    pltpu.sync_copy(rows_v, out_hbm.at[pl.ds(base, _BPW)])


def _mm_body(x_ref, wt_ref, o_ref):
    o_ref[...] = lax.dot_general(
        x_ref[...], wt_ref[...], (((1,), (0,)), ((), ())),
        preferred_element_type=jnp.float32)


_MB = 1024

_mm = pl.pallas_call(
    _mm_body,
    grid=(_N // _MB,),
    in_specs=[
        pl.BlockSpec((_MB, _DIM), lambda i: (i, jnp.int32(0))),
        pl.BlockSpec((_DIM, _MODEL_DIM),
                     lambda i: (jnp.int32(0), jnp.int32(0))),
    ],
    out_specs=pl.BlockSpec((_MB, _MODEL_DIM), lambda i: (i, jnp.int32(0))),
    out_shape=jax.ShapeDtypeStruct((_N, _MODEL_DIM), jnp.float32),
)


def kernel(input_ids, table, w_proj):
    b, s = input_ids.shape
    ids = input_ids.astype(jnp.int32)
    prev = jnp.pad(ids[:, :-1], ((0, 0), (1, 0)))
    emb = _hash_gather(ids.reshape(-1), prev.reshape(-1),
                       table.astype(jnp.float32))
    out = _mm(emb, w_proj.astype(jnp.float32).T)
    return out.astype(w_proj.dtype).reshape(b, s, _MODEL_DIM)
